# baseline (speedup 1.0000x reference)
import jax
import jax.numpy as jnp
from jax.experimental import pallas as pl


_ROWS_PER_BLOCK = 1024


def _select_body(mask_ref, emb_ref, x_ref, o_ref):
    m = mask_ref[...] != 0
    o_ref[...] = jnp.where(m, emb_ref[...], x_ref[...])


def kernel(x, mask, mask_emb):
    b, t, d = x.shape
    rows = b * t
    x2 = x.reshape(rows, d)
    m2 = mask.astype(jnp.int32).reshape(rows, 1)
    emb2 = mask_emb.reshape(1, d)

    grid = (rows // _ROWS_PER_BLOCK,)
    out = pl.pallas_call(
        _select_body,
        grid=grid,
        in_specs=[
            pl.BlockSpec((_ROWS_PER_BLOCK, 1), lambda i: (i, 0)),
            pl.BlockSpec((1, d), lambda i: (0, 0)),
            pl.BlockSpec((_ROWS_PER_BLOCK, d), lambda i: (i, 0)),
        ],
        out_specs=pl.BlockSpec((_ROWS_PER_BLOCK, d), lambda i: (i, 0)),
        out_shape=jax.ShapeDtypeStruct((rows, d), x.dtype),
    )(m2, emb2, x2)
    return out.reshape(b, t, d)
